# baseline (speedup 1.0000x reference)
import functools

import jax
import jax.numpy as jnp
from jax import lax
from jax.experimental import pallas as pl
from jax.experimental.pallas import tpu as pltpu
from jax.experimental.pallas import tpu_sc as plsc

N = 10000
E = 320000
D = 128

NC = 2
NS = 16
NW = NC * NS

CHUNK = 128
CHUNKS = 80
E_PAD = NW * CHUNKS * CHUNK
TOTAL_CHUNKS = NW * CHUNKS
K0 = 80
K1 = 2 * CHUNKS - K0
KMAX = max(K0, K1)
NPAD = 10240
TILE_N = NPAD // NS

_mesh = plsc.VectorSubcoreMesh(core_axis_name="c", subcore_axis_name="s")


@functools.partial(
    pl.kernel,
    out_type=jax.ShapeDtypeStruct((NC, 2, NPAD), jnp.float32),
    mesh=_mesh,
    scratch_types=[
        pltpu.VMEM((CHUNKS, CHUNK), jnp.int32),
        pltpu.VMEM((CHUNKS, CHUNK), jnp.int32),
        pltpu.VMEM((CHUNK,), jnp.float32),
        pltpu.VMEM((TILE_N,), jnp.float32),
        pltpu.VMEM_SHARED((NPAD,), jnp.float32),
        pltpu.VMEM_SHARED((NPAD,), jnp.float32),
    ],
)
def _hist(src_hbm, dst_hbm, out_hbm, src_v, dst_v, ones_v, buf_v, ho_sh, hi_sh):
    c = lax.axis_index("c")
    s = lax.axis_index("s")
    wid = s * NC + c

    def fill_ones(i, carry):
        ones_v[pl.ds(i * 16, 16)] = jnp.ones((16,), jnp.float32)
        return carry

    lax.fori_loop(0, CHUNK // 16, fill_ones, 0)

    def fill_zeros(i, carry):
        buf_v[pl.ds(i * 16, 16)] = jnp.zeros((16,), jnp.float32)
        return carry

    lax.fori_loop(0, TILE_N // 16, fill_zeros, 0)

    pltpu.sync_copy(buf_v, ho_sh.at[pl.ds(s * TILE_N, TILE_N)])
    pltpu.sync_copy(buf_v, hi_sh.at[pl.ds(s * TILE_N, TILE_N)])
    plsc.subcore_barrier()

    pltpu.sync_copy(src_hbm.at[pl.ds(wid * CHUNKS, CHUNKS)], src_v)
    pltpu.sync_copy(dst_hbm.at[pl.ds(wid * CHUNKS, CHUNKS)], dst_v)

    def body(j, carry):
        pltpu.sync_copy(ones_v, ho_sh.at[src_v.at[j]], add=True)
        pltpu.sync_copy(ones_v, hi_sh.at[dst_v.at[j]], add=True)
        return carry

    lax.fori_loop(0, CHUNKS, body, 0)
    plsc.subcore_barrier()

    pltpu.sync_copy(ho_sh.at[pl.ds(s * TILE_N, TILE_N)], buf_v)
    pltpu.sync_copy(buf_v, out_hbm.at[c, 0, pl.ds(s * TILE_N, TILE_N)])
    pltpu.sync_copy(hi_sh.at[pl.ds(s * TILE_N, TILE_N)], buf_v)
    pltpu.sync_copy(buf_v, out_hbm.at[c, 1, pl.ds(s * TILE_N, TILE_N)])


def _prep_body(deg_ref, feat_ref, w_ref, feat2_ref, normr_ref):
    deg = deg_ref[...]
    deg_out = deg[0, 0] + deg[1, 0]
    deg_in = deg[0, 1] + deg[1, 1]
    norm_l = lax.rsqrt(jnp.maximum(deg_out, 1.0))
    normr_ref[...] = lax.rsqrt(jnp.maximum(deg_in, 1.0))
    scaled = feat_ref[...] * norm_l
    f2 = lax.dot_general(
        scaled, w_ref[...], (((1,), (1,)), ((), ())),
        preferred_element_type=jnp.float32)
    feat2_ref[:NPAD] = f2
    feat2_ref[NPAD:] = f2


_prep = pl.pallas_call(
    _prep_body,
    out_shape=[
        jax.ShapeDtypeStruct((NC * NPAD, D), jnp.float32),
        jax.ShapeDtypeStruct((NPAD, 1), jnp.float32),
    ],
)


@functools.partial(
    pl.kernel,
    out_type=jax.ShapeDtypeStruct((NC, NPAD, D), jnp.float32),
    mesh=_mesh,
    scratch_types=[
        pltpu.VMEM((KMAX, CHUNK), jnp.int32),
        pltpu.VMEM((KMAX, CHUNK), jnp.int32),
        pltpu.VMEM((CHUNK, D), jnp.float32),
        pltpu.VMEM((32, D), jnp.float32),
        pltpu.VMEM_SHARED((NPAD, D), jnp.float32),
        pltpu.SemaphoreType.DMA,
    ],
)
def _spmm(feat2_hbm, src_hbm, dst_hbm, out_hbm, src_v, dst_v, rows_v, buf_v,
          acc_sh, sem):
    c = lax.axis_index("c")
    s = lax.axis_index("s")
    base = jnp.where(c == 0, s * K0, NS * K0 + s * K1)
    count = jnp.where(c == 0, K0, K1)

    def fill_zeros(i, carry):
        r = i // (D // 16)
        k = i % (D // 16)
        buf_v[r, pl.ds(k * 16, 16)] = jnp.zeros((16,), jnp.float32)
        return carry

    lax.fori_loop(0, 32 * (D // 16), fill_zeros, 0)

    def zero_acc(i, carry):
        pltpu.sync_copy(buf_v, acc_sh.at[pl.ds(s * TILE_N + i * 32, 32)])
        return carry

    lax.fori_loop(0, TILE_N // 32, zero_acc, 0)
    plsc.subcore_barrier()

    pltpu.sync_copy(src_hbm.at[pl.ds(base, KMAX)], src_v)
    pltpu.sync_copy(dst_hbm.at[pl.ds(base, KMAX)], dst_v)

    def body(j, carry):
        pltpu.async_copy(feat2_hbm.at[src_v.at[j]], rows_v, sem).wait()
        pltpu.sync_copy(rows_v, acc_sh.at[dst_v.at[j]], add=True)
        return carry

    lax.fori_loop(0, count, body, 0)
    plsc.subcore_barrier()

    def writeback(i, carry):
        pltpu.sync_copy(acc_sh.at[pl.ds(s * TILE_N + i * 32, 32)], buf_v)
        pltpu.sync_copy(buf_v, out_hbm.at[c, pl.ds(s * TILE_N + i * 32, 32)])
        return carry

    lax.fori_loop(0, TILE_N // 32, writeback, 0)


def _finish_body(p_ref, normr_ref, bias_ref, out_ref):
    p = p_ref[0] + p_ref[1]
    out_ref[...] = p[:N] * normr_ref[:N] + bias_ref[...]


_finish = pl.pallas_call(
    _finish_body,
    out_shape=jax.ShapeDtypeStruct((N, D), jnp.float32),
)


def kernel(feat, edge_index, weight, bias):
    feat_pad = jnp.pad(feat, ((0, NPAD - N), (0, 0)))
    pad_e = E_PAD - E
    src = jnp.concatenate([edge_index[0], jnp.full((pad_e,), N, jnp.int32)])
    dst = jnp.concatenate([edge_index[1], jnp.full((pad_e,), N, jnp.int32)])
    src = src.reshape(TOTAL_CHUNKS, CHUNK)
    dst = dst.reshape(TOTAL_CHUNKS, CHUNK)
    chunk_core = (jnp.arange(TOTAL_CHUNKS, dtype=jnp.int32)
                  >= NS * K0).astype(jnp.int32)
    src_off = src + chunk_core[:, None] * NPAD

    deg = _hist(src, dst)
    feat2, norm_r = _prep(deg[..., None], feat_pad, weight)
    partials = _spmm(feat2, src_off, dst)
    out = _finish(partials, norm_r, bias.reshape(1, D))
    return out


# speedup vs baseline: 1.0004x; 1.0004x over previous
import functools

import jax
import jax.numpy as jnp
from jax import lax
from jax.experimental import pallas as pl
from jax.experimental.pallas import tpu as pltpu
from jax.experimental.pallas import tpu_sc as plsc

N = 10000
E = 320000
D = 128

NC = 2
NS = 16
NW = NC * NS

CHUNK = 128
CHUNKS = 80
E_PAD = NW * CHUNKS * CHUNK
TOTAL_CHUNKS = NW * CHUNKS
K0 = 80
K1 = 2 * CHUNKS - K0
KMAX = max(K0, K1)
NPAD = 10240
TILE_N = NPAD // NS

_mesh = plsc.VectorSubcoreMesh(core_axis_name="c", subcore_axis_name="s")


@functools.partial(
    pl.kernel,
    out_type=jax.ShapeDtypeStruct((NC, 2, NPAD), jnp.float32),
    mesh=_mesh,
    scratch_types=[
        pltpu.VMEM((CHUNKS, CHUNK), jnp.int32),
        pltpu.VMEM((CHUNKS, CHUNK), jnp.int32),
        pltpu.VMEM((CHUNK,), jnp.float32),
        pltpu.VMEM((TILE_N,), jnp.float32),
        pltpu.VMEM_SHARED((NPAD,), jnp.float32),
        pltpu.VMEM_SHARED((NPAD,), jnp.float32),
    ],
)
def _hist(src_hbm, dst_hbm, out_hbm, src_v, dst_v, ones_v, buf_v, ho_sh, hi_sh):
    c = lax.axis_index("c")
    s = lax.axis_index("s")
    wid = s * NC + c

    def fill_ones(i, carry):
        ones_v[pl.ds(i * 16, 16)] = jnp.ones((16,), jnp.float32)
        return carry

    lax.fori_loop(0, CHUNK // 16, fill_ones, 0)

    def fill_zeros(i, carry):
        buf_v[pl.ds(i * 16, 16)] = jnp.zeros((16,), jnp.float32)
        return carry

    lax.fori_loop(0, TILE_N // 16, fill_zeros, 0)

    pltpu.sync_copy(buf_v, ho_sh.at[pl.ds(s * TILE_N, TILE_N)])
    pltpu.sync_copy(buf_v, hi_sh.at[pl.ds(s * TILE_N, TILE_N)])
    plsc.subcore_barrier()

    pltpu.sync_copy(src_hbm.at[pl.ds(wid * CHUNKS, CHUNKS)], src_v)
    pltpu.sync_copy(dst_hbm.at[pl.ds(wid * CHUNKS, CHUNKS)], dst_v)

    def body(j, carry):
        pltpu.sync_copy(ones_v, ho_sh.at[src_v.at[j]], add=True)
        pltpu.sync_copy(ones_v, hi_sh.at[dst_v.at[j]], add=True)
        return carry

    lax.fori_loop(0, CHUNKS, body, 0)
    plsc.subcore_barrier()

    pltpu.sync_copy(ho_sh.at[pl.ds(s * TILE_N, TILE_N)], buf_v)
    pltpu.sync_copy(buf_v, out_hbm.at[c, 0, pl.ds(s * TILE_N, TILE_N)])
    pltpu.sync_copy(hi_sh.at[pl.ds(s * TILE_N, TILE_N)], buf_v)
    pltpu.sync_copy(buf_v, out_hbm.at[c, 1, pl.ds(s * TILE_N, TILE_N)])


def _prep_body(deg_ref, feat_ref, w_ref, feat2_ref, normr_ref):
    deg = deg_ref[...]
    deg_out = deg[0, 0] + deg[1, 0]
    deg_in = deg[0, 1] + deg[1, 1]
    norm_l = lax.rsqrt(jnp.maximum(deg_out, 1.0))
    normr_ref[...] = lax.rsqrt(jnp.maximum(deg_in, 1.0))
    scaled = feat_ref[...] * norm_l
    f2 = lax.dot_general(
        scaled, w_ref[...], (((1,), (1,)), ((), ())),
        preferred_element_type=jnp.float32)
    feat2_ref[:NPAD] = f2
    feat2_ref[NPAD:] = f2


_prep = pl.pallas_call(
    _prep_body,
    out_shape=[
        jax.ShapeDtypeStruct((NC * NPAD, D), jnp.float32),
        jax.ShapeDtypeStruct((NPAD, 1), jnp.float32),
    ],
)


@functools.partial(
    pl.kernel,
    out_type=jax.ShapeDtypeStruct((NC, NPAD, D), jnp.float32),
    mesh=_mesh,
    scratch_types=[
        pltpu.VMEM((KMAX, CHUNK), jnp.int32),
        pltpu.VMEM((KMAX, CHUNK), jnp.int32),
        pltpu.VMEM((CHUNK, D), jnp.float32),
        pltpu.VMEM((32, D), jnp.float32),
        pltpu.VMEM_SHARED((NPAD, D), jnp.float32),
        pltpu.SemaphoreType.DMA,
    ],
)
def _spmm(feat2_hbm, src_hbm, dst_hbm, out_hbm, src_v, dst_v, rows_v, buf_v,
          acc_sh, sem):
    c = lax.axis_index("c")
    s = lax.axis_index("s")
    base = c * (NS * K0) + s * K0

    def fill_zeros(i, carry):
        r = i // (D // 16)
        k = i % (D // 16)
        buf_v[r, pl.ds(k * 16, 16)] = jnp.zeros((16,), jnp.float32)
        return carry

    lax.fori_loop(0, 32 * (D // 16), fill_zeros, 0)

    def zero_acc(i, carry):
        pltpu.sync_copy(buf_v, acc_sh.at[pl.ds(s * TILE_N + i * 32, 32)])
        return carry

    lax.fori_loop(0, TILE_N // 32, zero_acc, 0)
    plsc.subcore_barrier()

    pltpu.sync_copy(src_hbm.at[pl.ds(base, K0)], src_v)
    pltpu.sync_copy(dst_hbm.at[pl.ds(base, K0)], dst_v)

    def body(j, carry):
        pltpu.async_copy(feat2_hbm.at[src_v.at[j]], rows_v, sem).wait()
        pltpu.sync_copy(rows_v, acc_sh.at[dst_v.at[j]], add=True)
        return carry

    lax.fori_loop(0, K0, body, 0)
    plsc.subcore_barrier()

    def writeback(i, carry):
        pltpu.sync_copy(acc_sh.at[pl.ds(s * TILE_N + i * 32, 32)], buf_v)
        pltpu.sync_copy(buf_v, out_hbm.at[c, pl.ds(s * TILE_N + i * 32, 32)])
        return carry

    lax.fori_loop(0, TILE_N // 32, writeback, 0)


def _finish_body(p_ref, normr_ref, bias_ref, out_ref):
    p = p_ref[0] + p_ref[1]
    out_ref[...] = p[:N] * normr_ref[:N] + bias_ref[...]


_finish = pl.pallas_call(
    _finish_body,
    out_shape=jax.ShapeDtypeStruct((N, D), jnp.float32),
)


def kernel(feat, edge_index, weight, bias):
    feat_pad = jnp.pad(feat, ((0, NPAD - N), (0, 0)))
    pad_e = E_PAD - E
    src = jnp.concatenate([edge_index[0], jnp.full((pad_e,), N, jnp.int32)])
    dst = jnp.concatenate([edge_index[1], jnp.full((pad_e,), N, jnp.int32)])
    src = src.reshape(TOTAL_CHUNKS, CHUNK)
    dst = dst.reshape(TOTAL_CHUNKS, CHUNK)
    chunk_core = (jnp.arange(TOTAL_CHUNKS, dtype=jnp.int32)
                  >= NS * K0).astype(jnp.int32)
    src_off = src + chunk_core[:, None] * NPAD

    deg = _hist(src, dst)
    feat2, norm_r = _prep(deg[..., None], feat_pad, weight)
    partials = _spmm(feat2, src_off, dst)
    out = _finish(partials, norm_r, bias.reshape(1, D))
    return out


# speedup vs baseline: 1.1551x; 1.1547x over previous
import functools

import jax
import jax.numpy as jnp
from jax import lax
from jax.experimental import pallas as pl
from jax.experimental.pallas import tpu as pltpu
from jax.experimental.pallas import tpu_sc as plsc

N = 10000
E = 320000
D = 128

NC = 2
NS = 16
NW = NC * NS

CHUNK = 128
CHUNKS = 80
E_PAD = NW * CHUNKS * CHUNK
TOTAL_CHUNKS = NW * CHUNKS
K0 = 80
K1 = 2 * CHUNKS - K0
KMAX = max(K0, K1)
NPAD = 10240
TILE_N = NPAD // NS

_mesh = plsc.VectorSubcoreMesh(core_axis_name="c", subcore_axis_name="s")


@functools.partial(
    pl.kernel,
    out_type=jax.ShapeDtypeStruct((NC, 2, NPAD), jnp.float32),
    mesh=_mesh,
    scratch_types=[
        pltpu.VMEM((CHUNKS, CHUNK), jnp.int32),
        pltpu.VMEM((CHUNKS, CHUNK), jnp.int32),
        pltpu.VMEM((CHUNK,), jnp.float32),
        pltpu.VMEM((TILE_N,), jnp.float32),
        pltpu.VMEM_SHARED((NPAD,), jnp.float32),
        pltpu.VMEM_SHARED((NPAD,), jnp.float32),
    ],
)
def _hist(src_hbm, dst_hbm, out_hbm, src_v, dst_v, ones_v, buf_v, ho_sh, hi_sh):
    c = lax.axis_index("c")
    s = lax.axis_index("s")
    wid = s * NC + c

    def fill_ones(i, carry):
        ones_v[pl.ds(i * 16, 16)] = jnp.ones((16,), jnp.float32)
        return carry

    lax.fori_loop(0, CHUNK // 16, fill_ones, 0)

    def fill_zeros(i, carry):
        buf_v[pl.ds(i * 16, 16)] = jnp.zeros((16,), jnp.float32)
        return carry

    lax.fori_loop(0, TILE_N // 16, fill_zeros, 0)

    pltpu.sync_copy(buf_v, ho_sh.at[pl.ds(s * TILE_N, TILE_N)])
    pltpu.sync_copy(buf_v, hi_sh.at[pl.ds(s * TILE_N, TILE_N)])
    plsc.subcore_barrier()

    pltpu.sync_copy(src_hbm.at[pl.ds(wid * CHUNKS, CHUNKS)], src_v)
    pltpu.sync_copy(dst_hbm.at[pl.ds(wid * CHUNKS, CHUNKS)], dst_v)

    def body(j, carry):
        pltpu.sync_copy(ones_v, ho_sh.at[src_v.at[j]], add=True)
        pltpu.sync_copy(ones_v, hi_sh.at[dst_v.at[j]], add=True)
        return carry

    lax.fori_loop(0, CHUNKS, body, 0)
    plsc.subcore_barrier()

    pltpu.sync_copy(ho_sh.at[pl.ds(s * TILE_N, TILE_N)], buf_v)
    pltpu.sync_copy(buf_v, out_hbm.at[c, 0, pl.ds(s * TILE_N, TILE_N)])
    pltpu.sync_copy(hi_sh.at[pl.ds(s * TILE_N, TILE_N)], buf_v)
    pltpu.sync_copy(buf_v, out_hbm.at[c, 1, pl.ds(s * TILE_N, TILE_N)])


def _prep_body(deg_ref, feat_ref, w_ref, feat2_ref, normr_ref):
    deg = deg_ref[...]
    deg_out = deg[0, 0] + deg[1, 0]
    deg_in = deg[0, 1] + deg[1, 1]
    norm_l = lax.rsqrt(jnp.maximum(deg_out, 1.0))
    normr_ref[...] = lax.rsqrt(jnp.maximum(deg_in, 1.0))
    scaled = feat_ref[...] * norm_l
    feat2_ref[...] = lax.dot_general(
        scaled, w_ref[...], (((1,), (1,)), ((), ())),
        preferred_element_type=jnp.float32)


_prep = pl.pallas_call(
    _prep_body,
    out_shape=[
        jax.ShapeDtypeStruct((NPAD, D), jnp.float32),
        jax.ShapeDtypeStruct((NPAD, 1), jnp.float32),
    ],
)


@functools.partial(
    pl.kernel,
    out_type=jax.ShapeDtypeStruct((NC, NPAD, D), jnp.float32),
    mesh=_mesh,
    scratch_types=[
        pltpu.VMEM((KMAX, CHUNK), jnp.int32),
        pltpu.VMEM((KMAX, CHUNK), jnp.int32),
        pltpu.VMEM((CHUNK, D), jnp.float32),
        pltpu.VMEM((32, D), jnp.float32),
        pltpu.VMEM_SHARED((NPAD, D), jnp.float32),
        pltpu.SemaphoreType.DMA,
    ],
)
def _spmm(feat2_hbm, src_hbm, dst_hbm, out_hbm, src_v, dst_v, rows_v, buf_v,
          acc_sh, sem):
    c = lax.axis_index("c")
    s = lax.axis_index("s")
    base = c * (NS * K0) + s * K0

    def fill_zeros(i, carry):
        r = i // (D // 16)
        k = i % (D // 16)
        buf_v[r, pl.ds(k * 16, 16)] = jnp.zeros((16,), jnp.float32)
        return carry

    lax.fori_loop(0, 32 * (D // 16), fill_zeros, 0)

    def zero_acc(i, carry):
        pltpu.sync_copy(buf_v, acc_sh.at[pl.ds(s * TILE_N + i * 32, 32)])
        return carry

    lax.fori_loop(0, TILE_N // 32, zero_acc, 0)
    plsc.subcore_barrier()

    pltpu.sync_copy(src_hbm.at[pl.ds(base, K0)], src_v)
    pltpu.sync_copy(dst_hbm.at[pl.ds(base, K0)], dst_v)

    def body(j, carry):
        pltpu.async_copy(feat2_hbm.at[src_v.at[j]], rows_v, sem).wait()
        pltpu.sync_copy(rows_v, acc_sh.at[dst_v.at[j]], add=True)
        return carry

    lax.fori_loop(0, K0, body, 0)
    plsc.subcore_barrier()

    def writeback(i, carry):
        pltpu.sync_copy(acc_sh.at[pl.ds(s * TILE_N + i * 32, 32)], buf_v)
        pltpu.sync_copy(buf_v, out_hbm.at[c, pl.ds(s * TILE_N + i * 32, 32)])
        return carry

    lax.fori_loop(0, TILE_N // 32, writeback, 0)


def _finish_body(p_ref, normr_ref, bias_ref, out_ref):
    p = p_ref[0] + p_ref[1]
    out_ref[...] = p[:N] * normr_ref[:N] + bias_ref[...]


_finish = pl.pallas_call(
    _finish_body,
    out_shape=jax.ShapeDtypeStruct((N, D), jnp.float32),
)


def kernel(feat, edge_index, weight, bias):
    feat_pad = jnp.pad(feat, ((0, NPAD - N), (0, 0)))
    pad_e = E_PAD - E
    src = jnp.concatenate([edge_index[0], jnp.full((pad_e,), N, jnp.int32)])
    dst = jnp.concatenate([edge_index[1], jnp.full((pad_e,), N, jnp.int32)])
    src = src.reshape(TOTAL_CHUNKS, CHUNK)
    dst = dst.reshape(TOTAL_CHUNKS, CHUNK)

    deg = _hist(src, dst)
    feat2, norm_r = _prep(deg[..., None], feat_pad, weight)
    partials = _spmm(feat2, src, dst)
    out = _finish(partials, norm_r, bias.reshape(1, D))
    return out


# speedup vs baseline: 1.5594x; 1.3500x over previous
import functools

import jax
import jax.numpy as jnp
from jax import lax
from jax.experimental import pallas as pl
from jax.experimental.pallas import tpu as pltpu
from jax.experimental.pallas import tpu_sc as plsc

N = 10000
E = 320000
D = 128

NC = 2
NS = 16
NW = NC * NS

CHUNK = 128
CHUNKS = 79
E_PAD = NW * CHUNKS * CHUNK
NPAD = 10240
TILE_N = NPAD // NS

_mesh = plsc.VectorSubcoreMesh(core_axis_name="c", subcore_axis_name="s")


@functools.partial(
    pl.kernel,
    out_type=jax.ShapeDtypeStruct((NC, 2, NPAD), jnp.float32),
    mesh=_mesh,
    scratch_types=[
        pltpu.VMEM((CHUNKS, CHUNK), jnp.int32),
        pltpu.VMEM((CHUNKS, CHUNK), jnp.int32),
        pltpu.VMEM((CHUNK,), jnp.float32),
        pltpu.VMEM((TILE_N,), jnp.float32),
        pltpu.VMEM_SHARED((NPAD,), jnp.float32),
        pltpu.VMEM_SHARED((NPAD,), jnp.float32),
    ],
)
def _hist(src_hbm, dst_hbm, out_hbm, src_v, dst_v, ones_v, buf_v, ho_sh, hi_sh):
    c = lax.axis_index("c")
    s = lax.axis_index("s")
    wid = s * NC + c

    def fill_ones(i, carry):
        ones_v[pl.ds(i * 16, 16)] = jnp.ones((16,), jnp.float32)
        return carry

    lax.fori_loop(0, CHUNK // 16, fill_ones, 0)

    def fill_zeros(i, carry):
        buf_v[pl.ds(i * 16, 16)] = jnp.zeros((16,), jnp.float32)
        return carry

    lax.fori_loop(0, TILE_N // 16, fill_zeros, 0)

    pltpu.sync_copy(buf_v, ho_sh.at[pl.ds(s * TILE_N, TILE_N)])
    pltpu.sync_copy(buf_v, hi_sh.at[pl.ds(s * TILE_N, TILE_N)])
    plsc.subcore_barrier()

    pltpu.sync_copy(src_hbm.at[wid], src_v)
    pltpu.sync_copy(dst_hbm.at[wid], dst_v)

    def body(j, carry):
        pltpu.sync_copy(ones_v, ho_sh.at[src_v.at[j]], add=True)
        pltpu.sync_copy(ones_v, hi_sh.at[dst_v.at[j]], add=True)
        return carry

    lax.fori_loop(0, CHUNKS, body, 0)
    plsc.subcore_barrier()

    pltpu.sync_copy(ho_sh.at[pl.ds(s * TILE_N, TILE_N)], buf_v)
    pltpu.sync_copy(buf_v, out_hbm.at[c, 0, pl.ds(s * TILE_N, TILE_N)])
    pltpu.sync_copy(hi_sh.at[pl.ds(s * TILE_N, TILE_N)], buf_v)
    pltpu.sync_copy(buf_v, out_hbm.at[c, 1, pl.ds(s * TILE_N, TILE_N)])


def _mm_body(feat_ref, w_ref, g_ref):
    g_ref[...] = lax.dot_general(
        feat_ref[...], w_ref[...], (((1,), (1,)), ((), ())),
        preferred_element_type=jnp.float32)


_mm = pl.pallas_call(
    _mm_body,
    out_shape=jax.ShapeDtypeStruct((NPAD, D), jnp.float32),
)


def _prep_body(deg_ref, g_ref, feat2_ref, normr_ref):
    deg = deg_ref[...]
    deg_out = deg[0, 0] + deg[1, 0]
    deg_in = deg[0, 1] + deg[1, 1]
    norm_l = lax.rsqrt(jnp.maximum(deg_out, 1.0))
    normr_ref[...] = lax.rsqrt(jnp.maximum(deg_in, 1.0))
    feat2_ref[...] = g_ref[...] * norm_l


_prep = pl.pallas_call(
    _prep_body,
    out_shape=[
        jax.ShapeDtypeStruct((NPAD, D), jnp.float32),
        jax.ShapeDtypeStruct((NPAD, 1), jnp.float32),
    ],
)


@functools.partial(
    pl.kernel,
    out_type=jax.ShapeDtypeStruct((NC, NPAD, D), jnp.float32),
    mesh=_mesh,
    scratch_types=[
        pltpu.VMEM((CHUNKS, CHUNK), jnp.int32),
        pltpu.VMEM((CHUNKS, CHUNK), jnp.int32),
        pltpu.VMEM((CHUNK, D), jnp.float32),
        pltpu.VMEM((64, D), jnp.float32),
        pltpu.VMEM_SHARED((NPAD, D), jnp.float32),
        pltpu.SemaphoreType.DMA,
    ],
)
def _spmm(feat2_hbm, src_hbm, dst_hbm, out_hbm, src_v, dst_v, rows_v, buf_v,
          acc_sh, sem):
    c = lax.axis_index("c")
    s = lax.axis_index("s")
    wid = s * NC + c

    def fill_zeros(i, carry):
        r = i // (D // 16)
        k = i % (D // 16)
        buf_v[r, pl.ds(k * 16, 16)] = jnp.zeros((16,), jnp.float32)
        return carry

    lax.fori_loop(0, 64 * (D // 16), fill_zeros, 0)

    def zero_acc(i, carry):
        pltpu.sync_copy(buf_v, acc_sh.at[pl.ds(s * TILE_N + i * 64, 64)])
        return carry

    lax.fori_loop(0, TILE_N // 64, zero_acc, 0)
    plsc.subcore_barrier()

    pltpu.sync_copy(src_hbm.at[wid], src_v)
    pltpu.sync_copy(dst_hbm.at[wid], dst_v)

    def body(j, carry):
        pltpu.async_copy(feat2_hbm.at[src_v.at[j]], rows_v, sem).wait()
        pltpu.sync_copy(rows_v, acc_sh.at[dst_v.at[j]], add=True)
        return carry

    lax.fori_loop(0, CHUNKS, body, 0)
    plsc.subcore_barrier()

    def writeback(i, carry):
        pltpu.sync_copy(acc_sh.at[pl.ds(s * TILE_N + i * 64, 64)], buf_v)
        pltpu.sync_copy(buf_v, out_hbm.at[c, pl.ds(s * TILE_N + i * 64, 64)])
        return carry

    lax.fori_loop(0, TILE_N // 64, writeback, 0)


def _finish_body(p_ref, normr_ref, bias_ref, out_ref):
    p = p_ref[0] + p_ref[1]
    out_ref[...] = p[:N] * normr_ref[:N] + bias_ref[...]


_finish = pl.pallas_call(
    _finish_body,
    out_shape=jax.ShapeDtypeStruct((N, D), jnp.float32),
)


def kernel(feat, edge_index, weight, bias):
    feat_pad = jnp.pad(feat, ((0, NPAD - N), (0, 0)))
    pad_e = E_PAD - E
    src = jnp.concatenate([edge_index[0], jnp.full((pad_e,), N, jnp.int32)])
    dst = jnp.concatenate([edge_index[1], jnp.full((pad_e,), N, jnp.int32)])
    src = src.reshape(NW, CHUNKS, CHUNK)
    dst = dst.reshape(NW, CHUNKS, CHUNK)

    g = _mm(feat_pad, weight)
    deg = _hist(src, dst)
    feat2, norm_r = _prep(deg[..., None], g)
    partials = _spmm(feat2, src, dst)
    out = _finish(partials, norm_r, bias.reshape(1, D))
    return out


# speedup vs baseline: 1.6030x; 1.0280x over previous
import functools

import jax
import jax.numpy as jnp
from jax import lax
from jax.experimental import pallas as pl
from jax.experimental.pallas import tpu as pltpu
from jax.experimental.pallas import tpu_sc as plsc

N = 10000
E = 320000
D = 128

NC = 2
NS = 16
NW = NC * NS

CHUNK = 128
CHUNKS = 79
E_PAD = NW * CHUNKS * CHUNK
NPAD = 10240
TILE_N = NPAD // NS

_mesh = plsc.VectorSubcoreMesh(core_axis_name="c", subcore_axis_name="s")


@functools.partial(
    pl.kernel,
    out_type=jax.ShapeDtypeStruct((NC, 2, NPAD), jnp.float32),
    mesh=_mesh,
    scratch_types=[
        pltpu.VMEM((CHUNKS, CHUNK), jnp.int32),
        pltpu.VMEM((CHUNKS, CHUNK), jnp.int32),
        pltpu.VMEM((CHUNK,), jnp.float32),
        pltpu.VMEM((TILE_N,), jnp.float32),
        pltpu.VMEM_SHARED((NPAD,), jnp.float32),
        pltpu.VMEM_SHARED((NPAD,), jnp.float32),
    ],
)
def _hist(src_hbm, dst_hbm, out_hbm, src_v, dst_v, ones_v, buf_v, ho_sh, hi_sh):
    c = lax.axis_index("c")
    s = lax.axis_index("s")
    wid = s * NC + c

    def fill_ones(i, carry):
        ones_v[pl.ds(i * 16, 16)] = jnp.ones((16,), jnp.float32)
        return carry

    lax.fori_loop(0, CHUNK // 16, fill_ones, 0)

    def fill_zeros(i, carry):
        buf_v[pl.ds(i * 16, 16)] = jnp.zeros((16,), jnp.float32)
        return carry

    lax.fori_loop(0, TILE_N // 16, fill_zeros, 0)

    pltpu.sync_copy(buf_v, ho_sh.at[pl.ds(s * TILE_N, TILE_N)])
    pltpu.sync_copy(buf_v, hi_sh.at[pl.ds(s * TILE_N, TILE_N)])
    plsc.subcore_barrier()

    pltpu.sync_copy(src_hbm.at[wid], src_v)
    pltpu.sync_copy(dst_hbm.at[wid], dst_v)

    def body(j, carry):
        pltpu.sync_copy(ones_v, ho_sh.at[src_v.at[j]], add=True)
        pltpu.sync_copy(ones_v, hi_sh.at[dst_v.at[j]], add=True)
        return carry

    lax.fori_loop(0, CHUNKS, body, 0)
    plsc.subcore_barrier()

    pltpu.sync_copy(ho_sh.at[pl.ds(s * TILE_N, TILE_N)], buf_v)
    pltpu.sync_copy(buf_v, out_hbm.at[c, 0, pl.ds(s * TILE_N, TILE_N)])
    pltpu.sync_copy(hi_sh.at[pl.ds(s * TILE_N, TILE_N)], buf_v)
    pltpu.sync_copy(buf_v, out_hbm.at[c, 1, pl.ds(s * TILE_N, TILE_N)])


def _prep_body(deg_ref, feat_ref, w_ref, feat2_ref, normr_ref):
    deg = deg_ref[...]
    deg_out = deg[0, 0] + deg[1, 0]
    deg_in = deg[0, 1] + deg[1, 1]
    norm_l = lax.rsqrt(jnp.maximum(deg_out, 1.0))
    normr_ref[...] = lax.rsqrt(jnp.maximum(deg_in, 1.0))
    scaled = feat_ref[...] * norm_l
    feat2_ref[...] = lax.dot_general(
        scaled, w_ref[...], (((1,), (1,)), ((), ())),
        preferred_element_type=jnp.float32)


_prep = pl.pallas_call(
    _prep_body,
    out_shape=[
        jax.ShapeDtypeStruct((NPAD, D), jnp.float32),
        jax.ShapeDtypeStruct((NPAD, 1), jnp.float32),
    ],
)


@functools.partial(
    pl.kernel,
    out_type=jax.ShapeDtypeStruct((NC, NPAD, D), jnp.float32),
    mesh=_mesh,
    scratch_types=[
        pltpu.VMEM((CHUNKS, CHUNK), jnp.int32),
        pltpu.VMEM((CHUNKS, CHUNK), jnp.int32),
        pltpu.VMEM((CHUNK, D), jnp.float32),
        pltpu.VMEM((64, D), jnp.float32),
        pltpu.VMEM_SHARED((NPAD, D), jnp.float32),
        pltpu.SemaphoreType.DMA,
    ],
)
def _spmm(feat2_hbm, src_hbm, dst_hbm, out_hbm, src_v, dst_v, rows_v, buf_v,
          acc_sh, sem):
    c = lax.axis_index("c")
    s = lax.axis_index("s")
    wid = s * NC + c

    def fill_zeros(i, carry):
        r = i // (D // 16)
        k = i % (D // 16)
        buf_v[r, pl.ds(k * 16, 16)] = jnp.zeros((16,), jnp.float32)
        return carry

    lax.fori_loop(0, 64 * (D // 16), fill_zeros, 0)

    def zero_acc(i, carry):
        pltpu.sync_copy(buf_v, acc_sh.at[pl.ds(s * TILE_N + i * 64, 64)])
        return carry

    lax.fori_loop(0, TILE_N // 64, zero_acc, 0)
    plsc.subcore_barrier()

    pltpu.sync_copy(src_hbm.at[wid], src_v)
    pltpu.sync_copy(dst_hbm.at[wid], dst_v)

    def body(j, carry):
        pltpu.async_copy(feat2_hbm.at[src_v.at[j]], rows_v, sem).wait()
        pltpu.sync_copy(rows_v, acc_sh.at[dst_v.at[j]], add=True)
        return carry

    lax.fori_loop(0, CHUNKS, body, 0)
    plsc.subcore_barrier()

    def writeback(i, carry):
        pltpu.sync_copy(acc_sh.at[pl.ds(s * TILE_N + i * 64, 64)], buf_v)
        pltpu.sync_copy(buf_v, out_hbm.at[c, pl.ds(s * TILE_N + i * 64, 64)])
        return carry

    lax.fori_loop(0, TILE_N // 64, writeback, 0)


def _finish_body(p_ref, normr_ref, bias_ref, out_ref):
    p = p_ref[0] + p_ref[1]
    out_ref[...] = p[:N] * normr_ref[:N] + bias_ref[...]


_finish = pl.pallas_call(
    _finish_body,
    out_shape=jax.ShapeDtypeStruct((N, D), jnp.float32),
)


def kernel(feat, edge_index, weight, bias):
    feat_pad = jnp.pad(feat, ((0, NPAD - N), (0, 0)))
    pad_e = E_PAD - E
    src = jnp.concatenate([edge_index[0], jnp.full((pad_e,), N, jnp.int32)])
    dst = jnp.concatenate([edge_index[1], jnp.full((pad_e,), N, jnp.int32)])
    src = src.reshape(NW, CHUNKS, CHUNK)
    dst = dst.reshape(NW, CHUNKS, CHUNK)

    deg = _hist(src, dst)
    feat2, norm_r = _prep(deg[..., None], feat_pad, weight)
    partials = _spmm(feat2, src, dst)
    out = _finish(partials, norm_r, bias.reshape(1, D))
    return out


# speedup vs baseline: 1.6169x; 1.0087x over previous
import functools

import jax
import jax.numpy as jnp
from jax import lax
from jax.experimental import pallas as pl
from jax.experimental.pallas import tpu as pltpu
from jax.experimental.pallas import tpu_sc as plsc

N = 10000
E = 320000
D = 128

NC = 2
NS = 16
NW = NC * NS

CHUNK = 128
CHUNKS = 79
E_PAD = NW * CHUNKS * CHUNK
NPAD = 10240
TILE_N = NPAD // NS

_mesh = plsc.VectorSubcoreMesh(core_axis_name="c", subcore_axis_name="s")


@functools.partial(
    pl.kernel,
    out_type=jax.ShapeDtypeStruct((NC, 2, NPAD), jnp.float32),
    mesh=_mesh,
    scratch_types=[
        pltpu.VMEM((CHUNKS, CHUNK), jnp.int32),
        pltpu.VMEM((CHUNKS, CHUNK), jnp.int32),
        pltpu.VMEM((CHUNK,), jnp.float32),
        pltpu.VMEM((TILE_N,), jnp.float32),
        pltpu.VMEM_SHARED((NPAD,), jnp.float32),
        pltpu.VMEM_SHARED((NPAD,), jnp.float32),
    ],
)
def _hist(src_hbm, dst_hbm, out_hbm, src_v, dst_v, ones_v, buf_v, ho_sh, hi_sh):
    c = lax.axis_index("c")
    s = lax.axis_index("s")
    wid = s * NC + c

    def fill_ones(i, carry):
        ones_v[pl.ds(i * 16, 16)] = jnp.ones((16,), jnp.float32)
        return carry

    lax.fori_loop(0, CHUNK // 16, fill_ones, 0)

    def fill_zeros(i, carry):
        buf_v[pl.ds(i * 16, 16)] = jnp.zeros((16,), jnp.float32)
        return carry

    lax.fori_loop(0, TILE_N // 16, fill_zeros, 0)

    pltpu.sync_copy(buf_v, ho_sh.at[pl.ds(s * TILE_N, TILE_N)])
    pltpu.sync_copy(buf_v, hi_sh.at[pl.ds(s * TILE_N, TILE_N)])
    plsc.subcore_barrier()

    pltpu.sync_copy(src_hbm.at[wid], src_v)
    pltpu.sync_copy(dst_hbm.at[wid], dst_v)

    def body(j, carry):
        pltpu.sync_copy(ones_v, ho_sh.at[src_v.at[j]], add=True)
        pltpu.sync_copy(ones_v, hi_sh.at[dst_v.at[j]], add=True)
        return carry

    lax.fori_loop(0, CHUNKS, body, 0)
    plsc.subcore_barrier()

    pltpu.sync_copy(ho_sh.at[pl.ds(s * TILE_N, TILE_N)], buf_v)
    pltpu.sync_copy(buf_v, out_hbm.at[c, 0, pl.ds(s * TILE_N, TILE_N)])
    pltpu.sync_copy(hi_sh.at[pl.ds(s * TILE_N, TILE_N)], buf_v)
    pltpu.sync_copy(buf_v, out_hbm.at[c, 1, pl.ds(s * TILE_N, TILE_N)])


def _prep_body(deg_ref, feat_ref, w_ref, feat2_ref, normr_ref):
    deg = deg_ref[...]
    deg_out = deg[0, 0] + deg[1, 0]
    deg_in = deg[0, 1] + deg[1, 1]
    norm_l = lax.rsqrt(jnp.maximum(deg_out, 1.0))
    normr_ref[...] = lax.rsqrt(jnp.maximum(deg_in, 1.0))
    scaled = feat_ref[...] * norm_l
    feat2_ref[...] = lax.dot_general(
        scaled, w_ref[...], (((1,), (1,)), ((), ())),
        preferred_element_type=jnp.float32)


_prep = pl.pallas_call(
    _prep_body,
    out_shape=[
        jax.ShapeDtypeStruct((NPAD, D), jnp.float32),
        jax.ShapeDtypeStruct((NPAD, 1), jnp.float32),
    ],
)


@functools.partial(
    pl.kernel,
    out_type=jax.ShapeDtypeStruct((NC, NPAD, D), jnp.float32),
    mesh=_mesh,
    scratch_types=[
        pltpu.VMEM((CHUNKS, CHUNK), jnp.int32),
        pltpu.VMEM((CHUNKS, CHUNK), jnp.int32),
        pltpu.VMEM((CHUNK, D), jnp.float32),
        pltpu.VMEM((64, D), jnp.float32),
        pltpu.VMEM_SHARED((NPAD, D), jnp.float32),
        pltpu.SemaphoreType.DMA,
    ],
)
def _spmm(feat2_hbm, src_hbm, dst_hbm, out_hbm, src_v, dst_v, rows_v, buf_v,
          acc_sh, sem):
    c = lax.axis_index("c")
    s = lax.axis_index("s")
    wid = s * NC + c

    def fill_zeros(i, carry):
        r = i // (D // 16)
        k = i % (D // 16)
        buf_v[r, pl.ds(k * 16, 16)] = jnp.zeros((16,), jnp.float32)
        return carry

    lax.fori_loop(0, 64 * (D // 16), fill_zeros, 0)

    def zero_acc(i, carry):
        pltpu.sync_copy(buf_v, acc_sh.at[pl.ds(s * TILE_N + i * 64, 64)])
        return carry

    lax.fori_loop(0, TILE_N // 64, zero_acc, 0)
    plsc.subcore_barrier()

    pltpu.sync_copy(src_hbm.at[wid], src_v)
    pltpu.sync_copy(dst_hbm.at[wid], dst_v)

    def body(j, carry):
        pltpu.async_copy(feat2_hbm.at[src_v.at[j]], rows_v, sem).wait()
        pltpu.sync_copy(rows_v, acc_sh.at[dst_v.at[j]], add=True)
        return carry

    lax.fori_loop(0, CHUNKS, body, 0)
    plsc.subcore_barrier()

    pltpu.sync_copy(acc_sh.at[pl.ds(s * TILE_N, TILE_N)],
                    out_hbm.at[c, pl.ds(s * TILE_N, TILE_N)])


def _finish_body(p_ref, normr_ref, bias_ref, out_ref):
    p = p_ref[0] + p_ref[1]
    out_ref[...] = p[:N] * normr_ref[:N] + bias_ref[...]


_finish = pl.pallas_call(
    _finish_body,
    out_shape=jax.ShapeDtypeStruct((N, D), jnp.float32),
)


def kernel(feat, edge_index, weight, bias):
    feat_pad = jnp.pad(feat, ((0, NPAD - N), (0, 0)))
    pad_e = E_PAD - E
    src = jnp.concatenate([edge_index[0], jnp.full((pad_e,), N, jnp.int32)])
    dst = jnp.concatenate([edge_index[1], jnp.full((pad_e,), N, jnp.int32)])
    src = src.reshape(NW, CHUNKS, CHUNK)
    dst = dst.reshape(NW, CHUNKS, CHUNK)

    deg = _hist(src, dst)
    feat2, norm_r = _prep(deg[..., None], feat_pad, weight)
    partials = _spmm(feat2, src, dst)
    out = _finish(partials, norm_r, bias.reshape(1, D))
    return out
